# Initial kernel scaffold; baseline (speedup 1.0000x reference)
import jax
import jax.numpy as jnp
from jax.experimental import pallas as pl


def kernel(x, edge_index, edge_type, batch, node_type_ids, type_emb, W1, root1, b1, W2, root2, b2, node_w, node_b, graph_w, graph_b):
    raise NotImplementedError("write your pallas kernel here")


# speedup vs baseline: 5.9530x; 5.9530x over previous
import functools
import jax
import jax.numpy as jnp
from jax import lax
from jax.experimental import pallas as pl
from jax.experimental.pallas import tpu as pltpu
from jax.experimental.pallas import tpu_sc as plsc

N = 10000
E = 320000
BASE = 128
EMB = 32
HID = 256
R = 8
NT = 32
G = 64
SEG = R * N
NTILES = 16
EPT = E // NTILES
BATCH = 125
NB = EPT // BATCH
STRIPE = SEG // NTILES
NHALF = 2
NBH = NB // NHALF
NBK = 10
BLK = N // NBK


def _sc_agg_body(num_chunks, with_counts, *refs):
    CT = num_chunks + (1 if with_counts else 0)
    tabs = refs[:num_chunks]
    src_h, seg_h, zer_h, one_h = refs[num_chunks:num_chunks + 4]
    outs = refs[num_chunks + 4:num_chunks + 4 + CT]
    src_v, seg_v, rows_v, ones_v, acc, sem = refs[num_chunks + 4 + CT:]

    c = lax.axis_index("c")
    s = lax.axis_index("s")

    pltpu.sync_copy(one_h, ones_v)

    for ci in range(CT):
        is_cnt = with_counts and ci == CT - 1

        @pl.when(c == (ci % 2))
        def _(ci=ci, is_cnt=is_cnt):
            pltpu.sync_copy(zer_h, acc.at[pl.ds(s * STRIPE, STRIPE)])
            plsc.subcore_barrier()

            for hf in range(NHALF):
                if not is_cnt:
                    pltpu.sync_copy(src_h.at[s, pl.ds(hf * NBH, NBH)], src_v)
                pltpu.sync_copy(seg_h.at[s, pl.ds(hf * NBH, NBH)], seg_v)

                def bstep(b, carry):
                    segrow = seg_v.at[b]
                    if is_cnt:
                        pltpu.async_copy(ones_v, acc.at[segrow], sem,
                                         add=True).wait()
                    else:
                        pltpu.async_copy(tabs[ci].at[src_v.at[b]], rows_v,
                                         sem).wait()
                        pltpu.async_copy(rows_v, acc.at[segrow], sem,
                                         add=True).wait()
                    return carry

                lax.fori_loop(0, NBH, bstep, 0)
            plsc.subcore_barrier()
            pltpu.sync_copy(acc.at[pl.ds(s * STRIPE, STRIPE)],
                            outs[ci].at[pl.ds(s * STRIPE, STRIPE)])

    return None


def _make_sc_agg(num_chunks, with_counts):
    CT = num_chunks + (1 if with_counts else 0)
    mesh = plsc.VectorSubcoreMesh(core_axis_name="c", subcore_axis_name="s")
    out_type = tuple(jax.ShapeDtypeStruct((SEG, 16), jnp.float32)
                     for _ in range(CT))
    scratch = [
        pltpu.VMEM((NBH, BATCH), jnp.int32),
        pltpu.VMEM((NBH, BATCH), jnp.int32),
        pltpu.VMEM((BATCH, 16), jnp.float32),
        pltpu.VMEM((BATCH, 16), jnp.float32),
        pltpu.VMEM_SHARED((SEG, 16), jnp.float32),
        pltpu.SemaphoreType.DMA,
    ]
    return pl.kernel(
        functools.partial(_sc_agg_body, num_chunks, with_counts),
        out_type=out_type, mesh=mesh, scratch_types=scratch,
        compiler_params=pltpu.CompilerParams(use_tc_tiling_on_sc=False))


def _seg_body(et_ref, dst_ref, seg_ref):
    seg_ref[...] = et_ref[...] * N + dst_ref[...]


def _seg_ids(edge_type, dst):
    et = edge_type.reshape(2500, 128)
    d2 = dst.reshape(2500, 128)
    seg = pl.pallas_call(
        _seg_body,
        out_shape=jax.ShapeDtypeStruct((2500, 128), jnp.int32),
    )(et, d2)
    return seg.reshape(E)


def _h0_body(x_ref, ids_ref, emb_ref, h0_ref):
    ids = ids_ref[0, 0, :]
    oh = (ids[:, None] == lax.broadcasted_iota(jnp.int32, (BLK, NT), 1)
          ).astype(jnp.float32)
    h0_ref[:, :BASE] = x_ref[...]
    h0_ref[:, BASE:] = jnp.dot(oh, emb_ref[...],
                               preferred_element_type=jnp.float32)


def _h0_concat(x, node_type_ids, type_emb):
    ids3 = node_type_ids.reshape(NBK, 1, BLK)
    return pl.pallas_call(
        _h0_body,
        grid=(NBK,),
        in_specs=[
            pl.BlockSpec((BLK, BASE), lambda i: (i, 0)),
            pl.BlockSpec((1, 1, BLK), lambda i: (i, 0, 0)),
            pl.BlockSpec((NT, EMB), lambda i: (0, 0)),
        ],
        out_specs=pl.BlockSpec((BLK, BASE + EMB), lambda i: (i, 0)),
        out_shape=jax.ShapeDtypeStruct((N, BASE + EMB), jnp.float32),
    )(x, ids3, type_emb)


def _layer_body(D, h_ref, a_ref, cnt_ref, w_ref, root_ref, b_ref, out_ref):
    j = pl.program_id(1)

    @pl.when(j == 0)
    def _():
        out_ref[...] = jnp.zeros_like(out_ref)

    @pl.when(j < R)
    def _():
        inv = 1.0 / jnp.maximum(cnt_ref[0, 0, 0, :], 1.0)
        a = a_ref[0] * inv[:, None]
        out_ref[...] += jnp.dot(a, w_ref[0],
                                preferred_element_type=jnp.float32)

    @pl.when(j == R)
    def _():
        acc = out_ref[...] + jnp.dot(h_ref[...], root_ref[...],
                                     preferred_element_type=jnp.float32)
        out_ref[...] = jnp.maximum(acc + b_ref[...], 0.0)


def _rgcn_dense(h, A, cnt, W, root, b):
    D = h.shape[1]
    return pl.pallas_call(
        functools.partial(_layer_body, D),
        grid=(NBK, R + 1),
        in_specs=[
            pl.BlockSpec((BLK, D), lambda i, j: (i, 0)),
            pl.BlockSpec((1, BLK, D), lambda i, j: (jnp.minimum(j, R - 1), i, 0)),
            pl.BlockSpec((1, 1, 1, BLK),
                         lambda i, j: (jnp.minimum(j, R - 1), i, 0, 0)),
            pl.BlockSpec((1, D, HID), lambda i, j: (jnp.minimum(j, R - 1), 0, 0)),
            pl.BlockSpec((D, HID), lambda i, j: (0, 0)),
            pl.BlockSpec((1, HID), lambda i, j: (0, 0)),
        ],
        out_specs=pl.BlockSpec((BLK, HID), lambda i, j: (i, 0)),
        out_shape=jax.ShapeDtypeStruct((N, HID), jnp.float32),
    )(h, A, cnt.reshape(R, NBK, 1, BLK), W, root, b.reshape(1, HID))


def _final_body(h_ref, batch_ref, nw_ref, nb_ref, gw_ref, gb_ref,
                node_ref, graph_ref, gsum_ref, gcnt_ref):
    i = pl.program_id(0)
    h = h_ref[...]
    node_ref[...] = jnp.dot(h, nw_ref[...],
                            preferred_element_type=jnp.float32) + nb_ref[0, 0]
    b = batch_ref[0, 0, :]
    oh = (b[:, None] == lax.broadcasted_iota(jnp.int32, (BLK, G), 1)
          ).astype(jnp.float32)
    gs = lax.dot_general(oh, h, (((0,), (0,)), ((), ())),
                         preferred_element_type=jnp.float32)
    gc = jnp.sum(oh, axis=0)[:, None]

    @pl.when(i == 0)
    def _():
        gsum_ref[...] = jnp.zeros_like(gsum_ref)
        gcnt_ref[...] = jnp.zeros_like(gcnt_ref)

    gsum_ref[...] += gs
    gcnt_ref[...] += jnp.broadcast_to(gc, gcnt_ref.shape)

    @pl.when(i == NBK - 1)
    def _():
        ge = gsum_ref[...] / jnp.maximum(gcnt_ref[...], 1.0)
        graph_ref[...] = jnp.dot(ge, gw_ref[...],
                                 preferred_element_type=jnp.float32) + gb_ref[0, 0]


def _heads(h2, batch, node_w, node_b, graph_w, graph_b):
    batch3 = batch.reshape(NBK, 1, BLK)
    node, graph = pl.pallas_call(
        _final_body,
        grid=(NBK,),
        in_specs=[
            pl.BlockSpec((BLK, HID), lambda i: (i, 0)),
            pl.BlockSpec((1, 1, BLK), lambda i: (i, 0, 0)),
            pl.BlockSpec((HID, 1), lambda i: (0, 0)),
            pl.BlockSpec((1, 1), lambda i: (0, 0)),
            pl.BlockSpec((HID, 1), lambda i: (0, 0)),
            pl.BlockSpec((1, 1), lambda i: (0, 0)),
        ],
        out_specs=[
            pl.BlockSpec((BLK, 1), lambda i: (i, 0)),
            pl.BlockSpec((G, 1), lambda i: (0, 0)),
        ],
        out_shape=[
            jax.ShapeDtypeStruct((N, 1), jnp.float32),
            jax.ShapeDtypeStruct((G, 1), jnp.float32),
        ],
        scratch_shapes=[
            pltpu.VMEM((G, HID), jnp.float32),
            pltpu.VMEM((G, HID), jnp.float32),
        ],
    )(h2, batch3, node_w, node_b.reshape(1, 1), graph_w,
      graph_b.reshape(1, 1))
    return node[:, 0], graph[:, 0]


_agg1 = _make_sc_agg((BASE + EMB) // 16, True)
_agg2 = _make_sc_agg(HID // 16, False)


def _chunk_tables(h):
    C = h.shape[1] // 16
    hT = h.reshape(N, C, 16).transpose(1, 0, 2)
    return tuple(hT[i] for i in range(C))


def _assemble(outs, D):
    C = D // 16
    A = jnp.stack([o.reshape(R, N, 16) for o in outs[:C]], axis=2)
    return A.reshape(R, N, D)


def kernel(x, edge_index, edge_type, batch, node_type_ids, type_emb,
           W1, root1, b1, W2, root2, b2, node_w, node_b, graph_w, graph_b):
    src, dst = edge_index[0], edge_index[1]
    seg = _seg_ids(edge_type, dst)
    src3 = src.reshape(NTILES, NB, BATCH)
    seg3 = seg.reshape(NTILES, NB, BATCH)
    zeros = jnp.zeros((STRIPE, 16), jnp.float32)
    ones = jnp.ones((BATCH, 16), jnp.float32)

    h0 = _h0_concat(x, node_type_ids, type_emb)
    D0 = BASE + EMB

    outs1 = _agg1(*_chunk_tables(h0), src3, seg3, zeros, ones)
    cnt = outs1[-1][:, 0].reshape(R, N)
    A1 = _assemble(outs1, D0)
    h1 = _rgcn_dense(h0, A1, cnt, W1, root1, b1)

    outs2 = _agg2(*_chunk_tables(h1), src3, seg3, zeros, ones)
    A2 = _assemble(outs2, HID)
    h2 = _rgcn_dense(h1, A2, cnt, W2, root2, b2)

    return _heads(h2, batch, node_w, node_b, graph_w, graph_b)


# speedup vs baseline: 8.4779x; 1.4241x over previous
import functools
import jax
import jax.numpy as jnp
from jax import lax
from jax.experimental import pallas as pl
from jax.experimental.pallas import tpu as pltpu
from jax.experimental.pallas import tpu_sc as plsc

N = 10000
E = 320000
BASE = 128
EMB = 32
HID = 256
R = 8
NT = 32
G = 64
SEG = R * N
NTILES = 16
EPT = E // NTILES
BATCH = 125
NB = EPT // BATCH
STRIPE = SEG // NTILES
NHALF = 2
NBH = NB // NHALF
NDEEP = 4
NBK = 10
BLK = N // NBK


def _sc_agg_body(num_chunks, with_counts, *refs):
    CT = num_chunks + (1 if with_counts else 0)
    tabs = refs[:num_chunks]
    src_h, seg_h, zer_h, one_h = refs[num_chunks:num_chunks + 4]
    outs = refs[num_chunks + 4:num_chunks + 4 + CT]
    src_v, seg_v, rows_v, ones_v, acc, gsem, ssem = refs[num_chunks + 4 + CT:]

    c = lax.axis_index("c")
    s = lax.axis_index("s")

    pltpu.sync_copy(one_h, ones_v)

    for ci in range(CT):
        is_cnt = with_counts and ci == CT - 1

        @pl.when(c == (ci % 2))
        def _(ci=ci, is_cnt=is_cnt):
            pltpu.sync_copy(zer_h, acc.at[pl.ds(s * STRIPE, STRIPE)])
            plsc.subcore_barrier()

            for hf in range(NHALF):
                if not is_cnt:
                    pltpu.sync_copy(src_h.at[s, pl.ds(hf * NBH, NBH)], src_v)
                pltpu.sync_copy(seg_h.at[s, pl.ds(hf * NBH, NBH)], seg_v)

                def bstep(k, carry):
                    if is_cnt:
                        scat = [
                            pltpu.async_copy(
                                ones_v, acc.at[seg_v.at[k * NDEEP + j]],
                                ssem.at[j], add=True)
                            for j in range(NDEEP)]
                    else:
                        gat = [
                            pltpu.async_copy(
                                tabs[ci].at[src_v.at[k * NDEEP + j]],
                                rows_v.at[j], gsem.at[j])
                            for j in range(NDEEP)]
                        scat = []
                        for j in range(NDEEP):
                            gat[j].wait()
                            scat.append(pltpu.async_copy(
                                rows_v.at[j], acc.at[seg_v.at[k * NDEEP + j]],
                                ssem.at[j], add=True))
                    for d in scat:
                        d.wait()
                    return carry

                lax.fori_loop(0, NBH // NDEEP, bstep, 0)
            plsc.subcore_barrier()
            pltpu.sync_copy(acc.at[pl.ds(s * STRIPE, STRIPE)],
                            outs[ci].at[pl.ds(s * STRIPE, STRIPE)])

    return None


def _make_sc_agg(num_chunks, with_counts):
    CT = num_chunks + (1 if with_counts else 0)
    mesh = plsc.VectorSubcoreMesh(core_axis_name="c", subcore_axis_name="s")
    out_type = tuple(jax.ShapeDtypeStruct((SEG, 16), jnp.float32)
                     for _ in range(CT))
    scratch = [
        pltpu.VMEM((NBH, BATCH), jnp.int32),
        pltpu.VMEM((NBH, BATCH), jnp.int32),
        pltpu.VMEM((NDEEP, BATCH, 16), jnp.float32),
        pltpu.VMEM((BATCH, 16), jnp.float32),
        pltpu.VMEM_SHARED((SEG, 16), jnp.float32),
        pltpu.SemaphoreType.DMA((NDEEP,)),
        pltpu.SemaphoreType.DMA((NDEEP,)),
    ]
    return pl.kernel(
        functools.partial(_sc_agg_body, num_chunks, with_counts),
        out_type=out_type, mesh=mesh, scratch_types=scratch,
        compiler_params=pltpu.CompilerParams(use_tc_tiling_on_sc=False))


def _seg_body(et_ref, dst_ref, seg_ref):
    seg_ref[...] = et_ref[...] * N + dst_ref[...]


def _seg_ids(edge_type, dst):
    et = edge_type.reshape(2500, 128)
    d2 = dst.reshape(2500, 128)
    seg = pl.pallas_call(
        _seg_body,
        out_shape=jax.ShapeDtypeStruct((2500, 128), jnp.int32),
    )(et, d2)
    return seg.reshape(E)


def _h0_body(x_ref, ids_ref, emb_ref, h0_ref):
    ids = ids_ref[0, 0, :]
    oh = (ids[:, None] == lax.broadcasted_iota(jnp.int32, (BLK, NT), 1)
          ).astype(jnp.float32)
    h0_ref[:, :BASE] = x_ref[...]
    h0_ref[:, BASE:] = jnp.dot(oh, emb_ref[...],
                               preferred_element_type=jnp.float32)


def _h0_concat(x, node_type_ids, type_emb):
    ids3 = node_type_ids.reshape(NBK, 1, BLK)
    return pl.pallas_call(
        _h0_body,
        grid=(NBK,),
        in_specs=[
            pl.BlockSpec((BLK, BASE), lambda i: (i, 0)),
            pl.BlockSpec((1, 1, BLK), lambda i: (i, 0, 0)),
            pl.BlockSpec((NT, EMB), lambda i: (0, 0)),
        ],
        out_specs=pl.BlockSpec((BLK, BASE + EMB), lambda i: (i, 0)),
        out_shape=jax.ShapeDtypeStruct((N, BASE + EMB), jnp.float32),
    )(x, ids3, type_emb)


def _layer_body(D, h_ref, a_ref, cnt_ref, w_ref, root_ref, b_ref, out_ref):
    j = pl.program_id(1)

    @pl.when(j == 0)
    def _():
        out_ref[...] = jnp.zeros_like(out_ref)

    @pl.when(j < R)
    def _():
        inv = 1.0 / jnp.maximum(cnt_ref[0, 0, 0, :], 1.0)
        a = a_ref[0] * inv[:, None]
        out_ref[...] += jnp.dot(a, w_ref[0],
                                preferred_element_type=jnp.float32)

    @pl.when(j == R)
    def _():
        acc = out_ref[...] + jnp.dot(h_ref[...], root_ref[...],
                                     preferred_element_type=jnp.float32)
        out_ref[...] = jnp.maximum(acc + b_ref[...], 0.0)


def _rgcn_dense(h, A, cnt, W, root, b):
    D = h.shape[1]
    return pl.pallas_call(
        functools.partial(_layer_body, D),
        grid=(NBK, R + 1),
        in_specs=[
            pl.BlockSpec((BLK, D), lambda i, j: (i, 0)),
            pl.BlockSpec((1, BLK, D), lambda i, j: (jnp.minimum(j, R - 1), i, 0)),
            pl.BlockSpec((1, 1, 1, BLK),
                         lambda i, j: (jnp.minimum(j, R - 1), i, 0, 0)),
            pl.BlockSpec((1, D, HID), lambda i, j: (jnp.minimum(j, R - 1), 0, 0)),
            pl.BlockSpec((D, HID), lambda i, j: (0, 0)),
            pl.BlockSpec((1, HID), lambda i, j: (0, 0)),
        ],
        out_specs=pl.BlockSpec((BLK, HID), lambda i, j: (i, 0)),
        out_shape=jax.ShapeDtypeStruct((N, HID), jnp.float32),
    )(h, A, cnt.reshape(R, NBK, 1, BLK), W, root, b.reshape(1, HID))


def _final_body(h_ref, batch_ref, nw_ref, nb_ref, gw_ref, gb_ref,
                node_ref, graph_ref, gsum_ref, gcnt_ref):
    i = pl.program_id(0)
    h = h_ref[...]
    node_ref[...] = jnp.dot(h, nw_ref[...],
                            preferred_element_type=jnp.float32) + nb_ref[0, 0]
    b = batch_ref[0, 0, :]
    oh = (b[:, None] == lax.broadcasted_iota(jnp.int32, (BLK, G), 1)
          ).astype(jnp.float32)
    gs = lax.dot_general(oh, h, (((0,), (0,)), ((), ())),
                         preferred_element_type=jnp.float32)
    gc = jnp.sum(oh, axis=0)[:, None]

    @pl.when(i == 0)
    def _():
        gsum_ref[...] = jnp.zeros_like(gsum_ref)
        gcnt_ref[...] = jnp.zeros_like(gcnt_ref)

    gsum_ref[...] += gs
    gcnt_ref[...] += jnp.broadcast_to(gc, gcnt_ref.shape)

    @pl.when(i == NBK - 1)
    def _():
        ge = gsum_ref[...] / jnp.maximum(gcnt_ref[...], 1.0)
        graph_ref[...] = jnp.dot(ge, gw_ref[...],
                                 preferred_element_type=jnp.float32) + gb_ref[0, 0]


def _heads(h2, batch, node_w, node_b, graph_w, graph_b):
    batch3 = batch.reshape(NBK, 1, BLK)
    node, graph = pl.pallas_call(
        _final_body,
        grid=(NBK,),
        in_specs=[
            pl.BlockSpec((BLK, HID), lambda i: (i, 0)),
            pl.BlockSpec((1, 1, BLK), lambda i: (i, 0, 0)),
            pl.BlockSpec((HID, 1), lambda i: (0, 0)),
            pl.BlockSpec((1, 1), lambda i: (0, 0)),
            pl.BlockSpec((HID, 1), lambda i: (0, 0)),
            pl.BlockSpec((1, 1), lambda i: (0, 0)),
        ],
        out_specs=[
            pl.BlockSpec((BLK, 1), lambda i: (i, 0)),
            pl.BlockSpec((G, 1), lambda i: (0, 0)),
        ],
        out_shape=[
            jax.ShapeDtypeStruct((N, 1), jnp.float32),
            jax.ShapeDtypeStruct((G, 1), jnp.float32),
        ],
        scratch_shapes=[
            pltpu.VMEM((G, HID), jnp.float32),
            pltpu.VMEM((G, HID), jnp.float32),
        ],
    )(h2, batch3, node_w, node_b.reshape(1, 1), graph_w,
      graph_b.reshape(1, 1))
    return node[:, 0], graph[:, 0]


_agg1 = _make_sc_agg((BASE + EMB) // 16, True)
_agg2 = _make_sc_agg(HID // 16, False)


def _chunk_tables(h):
    C = h.shape[1] // 16
    hT = h.reshape(N, C, 16).transpose(1, 0, 2)
    return tuple(hT[i] for i in range(C))


def _assemble(outs, D):
    C = D // 16
    A = jnp.stack([o.reshape(R, N, 16) for o in outs[:C]], axis=2)
    return A.reshape(R, N, D)


def kernel(x, edge_index, edge_type, batch, node_type_ids, type_emb,
           W1, root1, b1, W2, root2, b2, node_w, node_b, graph_w, graph_b):
    src, dst = edge_index[0], edge_index[1]
    seg = _seg_ids(edge_type, dst)
    src3 = src.reshape(NTILES, NB, BATCH)
    seg3 = seg.reshape(NTILES, NB, BATCH)
    zeros = jnp.zeros((STRIPE, 16), jnp.float32)
    ones = jnp.ones((BATCH, 16), jnp.float32)

    h0 = _h0_concat(x, node_type_ids, type_emb)
    D0 = BASE + EMB

    outs1 = _agg1(*_chunk_tables(h0), src3, seg3, zeros, ones)
    cnt = outs1[-1][:, 0].reshape(R, N)
    A1 = _assemble(outs1, D0)
    h1 = _rgcn_dense(h0, A1, cnt, W1, root1, b1)

    outs2 = _agg2(*_chunk_tables(h1), src3, seg3, zeros, ones)
    A2 = _assemble(outs2, HID)
    h2 = _rgcn_dense(h1, A2, cnt, W2, root2, b2)

    return _heads(h2, batch, node_w, node_b, graph_w, graph_b)


# speedup vs baseline: 13.5726x; 1.6009x over previous
import functools
import jax
import jax.numpy as jnp
from jax import lax
from jax.experimental import pallas as pl
from jax.experimental.pallas import tpu as pltpu
from jax.experimental.pallas import tpu_sc as plsc

N = 10000
E = 320000
BASE = 128
EMB = 32
HID = 256
R = 8
NT = 32
G = 64
SEG = R * N
NTILES = 16
EPT = E // NTILES
BATCH = 125
NB = EPT // BATCH
STRIPE = SEG // NTILES
NHALF = 2
NBH = NB // NHALF
NDEEP = 4
NBK = 10
BLK = N // NBK


def _sc_agg_body(num_chunks, with_counts, *refs):
    CT = num_chunks + (1 if with_counts else 0)
    tab3, src_h, seg_h, zer_h, one_h = refs[:5]
    if with_counts:
        out_d, out_c = refs[5], refs[6]
        scr = refs[7:]
    else:
        out_d = refs[5]
        out_c = None
        scr = refs[6:]
    src_v, seg_v, rows_v, ones_v, acc, gsem, ssem = scr

    c = lax.axis_index("c")
    s = lax.axis_index("s")

    pltpu.sync_copy(one_h, ones_v)

    for ci in range(CT):
        is_cnt = with_counts and ci == CT - 1

        @pl.when(c == (ci % 2))
        def _(ci=ci, is_cnt=is_cnt):
            pltpu.sync_copy(zer_h, acc.at[pl.ds(s * STRIPE, STRIPE)])
            plsc.subcore_barrier()

            for hf in range(NHALF):
                if not is_cnt:
                    pltpu.sync_copy(src_h.at[s, pl.ds(hf * NBH, NBH)], src_v)
                pltpu.sync_copy(seg_h.at[s, pl.ds(hf * NBH, NBH)], seg_v)

                def bstep(k, carry):
                    if is_cnt:
                        scat = [
                            pltpu.async_copy(
                                ones_v, acc.at[seg_v.at[k * NDEEP + j]],
                                ssem.at[j], add=True)
                            for j in range(NDEEP)]
                    else:
                        gat = [
                            pltpu.async_copy(
                                tab3.at[ci].at[src_v.at[k * NDEEP + j]],
                                rows_v.at[j], gsem.at[j])
                            for j in range(NDEEP)]
                        scat = []
                        for j in range(NDEEP):
                            gat[j].wait()
                            scat.append(pltpu.async_copy(
                                rows_v.at[j], acc.at[seg_v.at[k * NDEEP + j]],
                                ssem.at[j], add=True))
                    for d in scat:
                        d.wait()
                    return carry

                lax.fori_loop(0, NBH // NDEEP, bstep, 0)
            plsc.subcore_barrier()
            if is_cnt:
                pltpu.sync_copy(acc.at[pl.ds(s * STRIPE, STRIPE)],
                                out_c.at[pl.ds(s * STRIPE, STRIPE)])
            else:
                pltpu.sync_copy(
                    acc.at[pl.ds(s * STRIPE, STRIPE)],
                    out_d.at[pl.ds(s * STRIPE, STRIPE), pl.ds(ci * 16, 16)])

    return None


def _make_sc_agg(num_chunks, with_counts):
    mesh = plsc.VectorSubcoreMesh(core_axis_name="c", subcore_axis_name="s")
    out_type = (jax.ShapeDtypeStruct((SEG, num_chunks * 16), jnp.float32),)
    if with_counts:
        out_type = out_type + (jax.ShapeDtypeStruct((SEG, 16), jnp.float32),)
    scratch = [
        pltpu.VMEM((NBH, BATCH), jnp.int32),
        pltpu.VMEM((NBH, BATCH), jnp.int32),
        pltpu.VMEM((NDEEP, BATCH, 16), jnp.float32),
        pltpu.VMEM((BATCH, 16), jnp.float32),
        pltpu.VMEM_SHARED((SEG, 16), jnp.float32),
        pltpu.SemaphoreType.DMA((NDEEP,)),
        pltpu.SemaphoreType.DMA((NDEEP,)),
    ]
    return pl.kernel(
        functools.partial(_sc_agg_body, num_chunks, with_counts),
        out_type=out_type, mesh=mesh, scratch_types=scratch,
        compiler_params=pltpu.CompilerParams(use_tc_tiling_on_sc=False))


def _seg_body(et_ref, dst_ref, seg_ref):
    seg_ref[...] = et_ref[...] * N + dst_ref[...]


def _seg_ids(edge_type, dst):
    et = edge_type.reshape(2500, 128)
    d2 = dst.reshape(2500, 128)
    seg = pl.pallas_call(
        _seg_body,
        out_shape=jax.ShapeDtypeStruct((2500, 128), jnp.int32),
    )(et, d2)
    return seg.reshape(E)


def _h0_body(x_ref, ids_ref, emb_ref, h0_ref):
    ids = ids_ref[0, 0, :]
    oh = (ids[:, None] == lax.broadcasted_iota(jnp.int32, (BLK, NT), 1)
          ).astype(jnp.float32)
    h0_ref[:, :BASE] = x_ref[...]
    h0_ref[:, BASE:] = jnp.dot(oh, emb_ref[...],
                               preferred_element_type=jnp.float32)


def _h0_concat(x, node_type_ids, type_emb):
    ids3 = node_type_ids.reshape(NBK, 1, BLK)
    return pl.pallas_call(
        _h0_body,
        grid=(NBK,),
        in_specs=[
            pl.BlockSpec((BLK, BASE), lambda i: (i, 0)),
            pl.BlockSpec((1, 1, BLK), lambda i: (i, 0, 0)),
            pl.BlockSpec((NT, EMB), lambda i: (0, 0)),
        ],
        out_specs=pl.BlockSpec((BLK, BASE + EMB), lambda i: (i, 0)),
        out_shape=jax.ShapeDtypeStruct((N, BASE + EMB), jnp.float32),
    )(x, ids3, type_emb)


def _layer_body(D, h_ref, a_ref, cnt_ref, w_ref, root_ref, b_ref, out_ref):
    j = pl.program_id(1)

    @pl.when(j == 0)
    def _():
        out_ref[...] = jnp.zeros_like(out_ref)

    @pl.when(j < R)
    def _():
        inv = 1.0 / jnp.maximum(cnt_ref[0, 0, 0, :], 1.0)
        a = a_ref[0] * inv[:, None]
        out_ref[...] += jnp.dot(a, w_ref[0],
                                preferred_element_type=jnp.float32)

    @pl.when(j == R)
    def _():
        acc = out_ref[...] + jnp.dot(h_ref[...], root_ref[...],
                                     preferred_element_type=jnp.float32)
        out_ref[...] = jnp.maximum(acc + b_ref[...], 0.0)


def _rgcn_dense(h, A, cnt, W, root, b):
    D = h.shape[1]
    return pl.pallas_call(
        functools.partial(_layer_body, D),
        grid=(NBK, R + 1),
        in_specs=[
            pl.BlockSpec((BLK, D), lambda i, j: (i, 0)),
            pl.BlockSpec((1, BLK, D), lambda i, j: (jnp.minimum(j, R - 1), i, 0)),
            pl.BlockSpec((1, 1, 1, BLK),
                         lambda i, j: (jnp.minimum(j, R - 1), i, 0, 0)),
            pl.BlockSpec((1, D, HID), lambda i, j: (jnp.minimum(j, R - 1), 0, 0)),
            pl.BlockSpec((D, HID), lambda i, j: (0, 0)),
            pl.BlockSpec((1, HID), lambda i, j: (0, 0)),
        ],
        out_specs=pl.BlockSpec((BLK, HID), lambda i, j: (i, 0)),
        out_shape=jax.ShapeDtypeStruct((N, HID), jnp.float32),
    )(h, A, cnt.reshape(R, NBK, 1, BLK), W, root, b.reshape(1, HID))


def _final_body(h_ref, batch_ref, nw_ref, nb_ref, gw_ref, gb_ref,
                node_ref, graph_ref, gsum_ref, gcnt_ref):
    i = pl.program_id(0)
    h = h_ref[...]
    node_ref[...] = jnp.dot(h, nw_ref[...],
                            preferred_element_type=jnp.float32) + nb_ref[0, 0]
    b = batch_ref[0, 0, :]
    oh = (b[:, None] == lax.broadcasted_iota(jnp.int32, (BLK, G), 1)
          ).astype(jnp.float32)
    gs = lax.dot_general(oh, h, (((0,), (0,)), ((), ())),
                         preferred_element_type=jnp.float32)
    gc = jnp.sum(oh, axis=0)[:, None]

    @pl.when(i == 0)
    def _():
        gsum_ref[...] = jnp.zeros_like(gsum_ref)
        gcnt_ref[...] = jnp.zeros_like(gcnt_ref)

    gsum_ref[...] += gs
    gcnt_ref[...] += jnp.broadcast_to(gc, gcnt_ref.shape)

    @pl.when(i == NBK - 1)
    def _():
        ge = gsum_ref[...] / jnp.maximum(gcnt_ref[...], 1.0)
        graph_ref[...] = jnp.dot(ge, gw_ref[...],
                                 preferred_element_type=jnp.float32) + gb_ref[0, 0]


def _heads(h2, batch, node_w, node_b, graph_w, graph_b):
    batch3 = batch.reshape(NBK, 1, BLK)
    node, graph = pl.pallas_call(
        _final_body,
        grid=(NBK,),
        in_specs=[
            pl.BlockSpec((BLK, HID), lambda i: (i, 0)),
            pl.BlockSpec((1, 1, BLK), lambda i: (i, 0, 0)),
            pl.BlockSpec((HID, 1), lambda i: (0, 0)),
            pl.BlockSpec((1, 1), lambda i: (0, 0)),
            pl.BlockSpec((HID, 1), lambda i: (0, 0)),
            pl.BlockSpec((1, 1), lambda i: (0, 0)),
        ],
        out_specs=[
            pl.BlockSpec((BLK, 1), lambda i: (i, 0)),
            pl.BlockSpec((G, 1), lambda i: (0, 0)),
        ],
        out_shape=[
            jax.ShapeDtypeStruct((N, 1), jnp.float32),
            jax.ShapeDtypeStruct((G, 1), jnp.float32),
        ],
        scratch_shapes=[
            pltpu.VMEM((G, HID), jnp.float32),
            pltpu.VMEM((G, HID), jnp.float32),
        ],
    )(h2, batch3, node_w, node_b.reshape(1, 1), graph_w,
      graph_b.reshape(1, 1))
    return node[:, 0], graph[:, 0]


_agg1 = _make_sc_agg((BASE + EMB) // 16, True)
_agg2 = _make_sc_agg(HID // 16, False)


def _chunk_tables(h):
    C = h.shape[1] // 16
    return h.reshape(N, C, 16).transpose(1, 0, 2)


def kernel(x, edge_index, edge_type, batch, node_type_ids, type_emb,
           W1, root1, b1, W2, root2, b2, node_w, node_b, graph_w, graph_b):
    src, dst = edge_index[0], edge_index[1]
    seg = _seg_ids(edge_type, dst)
    src3 = src.reshape(NTILES, NB, BATCH)
    seg3 = seg.reshape(NTILES, NB, BATCH)
    zeros = jnp.zeros((STRIPE, 16), jnp.float32)
    ones = jnp.ones((BATCH, 16), jnp.float32)

    h0 = _h0_concat(x, node_type_ids, type_emb)
    D0 = BASE + EMB

    A1flat, cntout = _agg1(_chunk_tables(h0), src3, seg3, zeros, ones)
    cnt = cntout[:, 0].reshape(R, N)
    A1 = A1flat.reshape(R, N, D0)
    h1 = _rgcn_dense(h0, A1, cnt, W1, root1, b1)

    (A2flat,) = _agg2(_chunk_tables(h1), src3, seg3, zeros, ones)
    A2 = A2flat.reshape(R, N, HID)
    h2 = _rgcn_dense(h1, A2, cnt, W2, root2, b2)

    return _heads(h2, batch, node_w, node_b, graph_w, graph_b)


# speedup vs baseline: 15.4099x; 1.1354x over previous
import functools
import jax
import jax.numpy as jnp
from jax import lax
from jax.experimental import pallas as pl
from jax.experimental.pallas import tpu as pltpu
from jax.experimental.pallas import tpu_sc as plsc

N = 10000
E = 320000
BASE = 128
EMB = 32
HID = 256
R = 8
NT = 32
G = 64
SEG = R * N
NTILES = 16
EPT = E // NTILES
BATCH = 125
NB = EPT // BATCH
STRIPE = SEG // NTILES
NHALF = 2
NBH = NB // NHALF
NDEEP = 8
NBK = 10
BLK = N // NBK


def _sc_agg_body(num_chunks, with_counts, *refs):
    CT = num_chunks + (1 if with_counts else 0)
    tab3, src_h, seg_h, zer_h, one_h = refs[:5]
    if with_counts:
        out_d, out_c = refs[5], refs[6]
        scr = refs[7:]
    else:
        out_d = refs[5]
        out_c = None
        scr = refs[6:]
    src_v, seg_v, rows_v, ones_v, acc, gsem, ssem = scr

    c = lax.axis_index("c")
    s = lax.axis_index("s")

    pltpu.sync_copy(one_h, ones_v)

    for ci in range(CT):
        is_cnt = with_counts and ci == CT - 1

        @pl.when(c == (ci % 2))
        def _(ci=ci, is_cnt=is_cnt):
            pltpu.sync_copy(zer_h, acc.at[pl.ds(s * STRIPE, STRIPE)])
            plsc.subcore_barrier()

            for hf in range(NHALF):
                if not is_cnt:
                    pltpu.sync_copy(src_h.at[s, pl.ds(hf * NBH, NBH)], src_v)
                pltpu.sync_copy(seg_h.at[s, pl.ds(hf * NBH, NBH)], seg_v)

                def bstep(k, carry):
                    if is_cnt:
                        scat = [
                            pltpu.async_copy(
                                ones_v, acc.at[seg_v.at[k * NDEEP + j]],
                                ssem.at[j], add=True)
                            for j in range(NDEEP)]
                    else:
                        gat = [
                            pltpu.async_copy(
                                tab3.at[ci].at[src_v.at[k * NDEEP + j]],
                                rows_v.at[j], gsem.at[j])
                            for j in range(NDEEP)]
                        scat = []
                        for j in range(NDEEP):
                            gat[j].wait()
                            scat.append(pltpu.async_copy(
                                rows_v.at[j], acc.at[seg_v.at[k * NDEEP + j]],
                                ssem.at[j], add=True))
                    for d in scat:
                        d.wait()
                    return carry

                lax.fori_loop(0, NBH // NDEEP, bstep, 0)
            plsc.subcore_barrier()
            if is_cnt:
                pltpu.sync_copy(acc.at[pl.ds(s * STRIPE, STRIPE)],
                                out_c.at[pl.ds(s * STRIPE, STRIPE)])
            else:
                pltpu.sync_copy(
                    acc.at[pl.ds(s * STRIPE, STRIPE)],
                    out_d.at[pl.ds(s * STRIPE, STRIPE), pl.ds(ci * 16, 16)])

    return None


def _make_sc_agg(num_chunks, with_counts):
    mesh = plsc.VectorSubcoreMesh(core_axis_name="c", subcore_axis_name="s")
    out_type = (jax.ShapeDtypeStruct((SEG, num_chunks * 16), jnp.float32),)
    if with_counts:
        out_type = out_type + (jax.ShapeDtypeStruct((SEG, 16), jnp.float32),)
    scratch = [
        pltpu.VMEM((NBH, BATCH), jnp.int32),
        pltpu.VMEM((NBH, BATCH), jnp.int32),
        pltpu.VMEM((NDEEP, BATCH, 16), jnp.float32),
        pltpu.VMEM((BATCH, 16), jnp.float32),
        pltpu.VMEM_SHARED((SEG, 16), jnp.float32),
        pltpu.SemaphoreType.DMA((NDEEP,)),
        pltpu.SemaphoreType.DMA((NDEEP,)),
    ]
    return pl.kernel(
        functools.partial(_sc_agg_body, num_chunks, with_counts),
        out_type=out_type, mesh=mesh, scratch_types=scratch,
        compiler_params=pltpu.CompilerParams(use_tc_tiling_on_sc=False))


def _seg_body(et_ref, dst_ref, seg_ref):
    seg_ref[...] = et_ref[...] * N + dst_ref[...]


def _seg_ids(edge_type, dst):
    et = edge_type.reshape(2500, 128)
    d2 = dst.reshape(2500, 128)
    seg = pl.pallas_call(
        _seg_body,
        out_shape=jax.ShapeDtypeStruct((2500, 128), jnp.int32),
    )(et, d2)
    return seg.reshape(E)


def _h0_body(x_ref, ids_ref, emb_ref, h0_ref):
    ids = ids_ref[0, 0, :]
    oh = (ids[:, None] == lax.broadcasted_iota(jnp.int32, (BLK, NT), 1)
          ).astype(jnp.float32)
    h0_ref[:, :BASE] = x_ref[...]
    h0_ref[:, BASE:] = jnp.dot(oh, emb_ref[...],
                               preferred_element_type=jnp.float32)


def _h0_concat(x, node_type_ids, type_emb):
    ids3 = node_type_ids.reshape(NBK, 1, BLK)
    return pl.pallas_call(
        _h0_body,
        grid=(NBK,),
        in_specs=[
            pl.BlockSpec((BLK, BASE), lambda i: (i, 0)),
            pl.BlockSpec((1, 1, BLK), lambda i: (i, 0, 0)),
            pl.BlockSpec((NT, EMB), lambda i: (0, 0)),
        ],
        out_specs=pl.BlockSpec((BLK, BASE + EMB), lambda i: (i, 0)),
        out_shape=jax.ShapeDtypeStruct((N, BASE + EMB), jnp.float32),
    )(x, ids3, type_emb)


def _layer_body(D, h_ref, a_ref, cnt_ref, w_ref, root_ref, b_ref, out_ref):
    j = pl.program_id(1)

    @pl.when(j == 0)
    def _():
        out_ref[...] = jnp.zeros_like(out_ref)

    @pl.when(j < R)
    def _():
        inv = 1.0 / jnp.maximum(cnt_ref[:, 0], 1.0)
        a = a_ref[...] * inv[:, None]
        out_ref[...] += jnp.dot(a, w_ref[0],
                                preferred_element_type=jnp.float32)

    @pl.when(j == R)
    def _():
        acc = out_ref[...] + jnp.dot(h_ref[...], root_ref[...],
                                     preferred_element_type=jnp.float32)
        out_ref[...] = jnp.maximum(acc + b_ref[...], 0.0)


def _rgcn_dense(h, A, cnt, W, root, b):
    D = h.shape[1]
    return pl.pallas_call(
        functools.partial(_layer_body, D),
        grid=(NBK, R + 1),
        in_specs=[
            pl.BlockSpec((BLK, D), lambda i, j: (i, 0)),
            pl.BlockSpec((BLK, D),
                         lambda i, j: (jnp.minimum(j, R - 1) * NBK + i, 0)),
            pl.BlockSpec((BLK, 16),
                         lambda i, j: (jnp.minimum(j, R - 1) * NBK + i, 0)),
            pl.BlockSpec((1, D, HID), lambda i, j: (jnp.minimum(j, R - 1), 0, 0)),
            pl.BlockSpec((D, HID), lambda i, j: (0, 0)),
            pl.BlockSpec((1, HID), lambda i, j: (0, 0)),
        ],
        out_specs=pl.BlockSpec((BLK, HID), lambda i, j: (i, 0)),
        out_shape=jax.ShapeDtypeStruct((N, HID), jnp.float32),
    )(h, A, cnt, W, root, b.reshape(1, HID))


def _final_body(h_ref, batch_ref, nw_ref, nb_ref, gw_ref, gb_ref,
                node_ref, graph_ref, gsum_ref, gcnt_ref):
    i = pl.program_id(0)
    h = h_ref[...]
    node_ref[...] = jnp.dot(h, nw_ref[...],
                            preferred_element_type=jnp.float32) + nb_ref[0, 0]
    b = batch_ref[0, 0, :]
    oh = (b[:, None] == lax.broadcasted_iota(jnp.int32, (BLK, G), 1)
          ).astype(jnp.float32)
    gs = lax.dot_general(oh, h, (((0,), (0,)), ((), ())),
                         preferred_element_type=jnp.float32)
    gc = jnp.sum(oh, axis=0)[:, None]

    @pl.when(i == 0)
    def _():
        gsum_ref[...] = jnp.zeros_like(gsum_ref)
        gcnt_ref[...] = jnp.zeros_like(gcnt_ref)

    gsum_ref[...] += gs
    gcnt_ref[...] += jnp.broadcast_to(gc, gcnt_ref.shape)

    @pl.when(i == NBK - 1)
    def _():
        ge = gsum_ref[...] / jnp.maximum(gcnt_ref[...], 1.0)
        graph_ref[...] = jnp.dot(ge, gw_ref[...],
                                 preferred_element_type=jnp.float32) + gb_ref[0, 0]


def _heads(h2, batch, node_w, node_b, graph_w, graph_b):
    batch3 = batch.reshape(NBK, 1, BLK)
    node, graph = pl.pallas_call(
        _final_body,
        grid=(NBK,),
        in_specs=[
            pl.BlockSpec((BLK, HID), lambda i: (i, 0)),
            pl.BlockSpec((1, 1, BLK), lambda i: (i, 0, 0)),
            pl.BlockSpec((HID, 1), lambda i: (0, 0)),
            pl.BlockSpec((1, 1), lambda i: (0, 0)),
            pl.BlockSpec((HID, 1), lambda i: (0, 0)),
            pl.BlockSpec((1, 1), lambda i: (0, 0)),
        ],
        out_specs=[
            pl.BlockSpec((BLK, 1), lambda i: (i, 0)),
            pl.BlockSpec((G, 1), lambda i: (0, 0)),
        ],
        out_shape=[
            jax.ShapeDtypeStruct((N, 1), jnp.float32),
            jax.ShapeDtypeStruct((G, 1), jnp.float32),
        ],
        scratch_shapes=[
            pltpu.VMEM((G, HID), jnp.float32),
            pltpu.VMEM((G, HID), jnp.float32),
        ],
    )(h2, batch3, node_w, node_b.reshape(1, 1), graph_w,
      graph_b.reshape(1, 1))
    return node[:, 0], graph[:, 0]


_agg1 = _make_sc_agg((BASE + EMB) // 16, True)
_agg2 = _make_sc_agg(HID // 16, False)


def _chunk_tables(h):
    C = h.shape[1] // 16
    return h.reshape(N, C, 16).transpose(1, 0, 2)


def kernel(x, edge_index, edge_type, batch, node_type_ids, type_emb,
           W1, root1, b1, W2, root2, b2, node_w, node_b, graph_w, graph_b):
    src, dst = edge_index[0], edge_index[1]
    seg = _seg_ids(edge_type, dst)
    src3 = src.reshape(NTILES, NB, BATCH)
    seg3 = seg.reshape(NTILES, NB, BATCH)
    zeros = jnp.zeros((STRIPE, 16), jnp.float32)
    ones = jnp.ones((BATCH, 16), jnp.float32)

    h0 = _h0_concat(x, node_type_ids, type_emb)
    D0 = BASE + EMB

    A1, cnt = _agg1(_chunk_tables(h0), src3, seg3, zeros, ones)
    h1 = _rgcn_dense(h0, A1, cnt, W1, root1, b1)

    (A2,) = _agg2(_chunk_tables(h1), src3, seg3, zeros, ones)
    h2 = _rgcn_dense(h1, A2, cnt, W2, root2, b2)

    return _heads(h2, batch, node_w, node_b, graph_w, graph_b)


# speedup vs baseline: 16.2465x; 1.0543x over previous
import functools
import jax
import jax.numpy as jnp
from jax import lax
from jax.experimental import pallas as pl
from jax.experimental.pallas import tpu as pltpu
from jax.experimental.pallas import tpu_sc as plsc

N = 10000
E = 320000
BASE = 128
EMB = 32
HID = 256
R = 8
NT = 32
G = 64
SEG = R * N
NTILES = 16
EPT = E // NTILES
BATCH = 125
NB = EPT // BATCH
STRIPE = SEG // NTILES
NHALF = 2
NBH = NB // NHALF
NDEEP = 8
NBK = 10
BLK = N // NBK


def _sc_agg_body(num_chunks, with_counts, *refs):
    CT = num_chunks + (1 if with_counts else 0)
    tab3, src_h, seg_h, zer_h, one_h = refs[:5]
    if with_counts:
        out_d, out_c = refs[5], refs[6]
        scr = refs[7:]
    else:
        out_d = refs[5]
        out_c = None
        scr = refs[6:]
    src_v, seg_v, rows_v, ones_v, acc, gsem, ssem = scr

    c = lax.axis_index("c")
    s = lax.axis_index("s")

    pltpu.sync_copy(one_h, ones_v)

    def drain(j):
        pltpu.make_async_copy(
            rows_v.at[j], acc.at[seg_v.at[0]], ssem.at[j]).wait()

    def chunk_body(ci, carry):
        @pl.when(c == lax.rem(ci, 2))
        def _():
            pltpu.sync_copy(zer_h, acc.at[pl.ds(s * STRIPE, STRIPE)])
            plsc.subcore_barrier()

            for hf in range(NHALF):
                pltpu.sync_copy(src_h.at[s, pl.ds(hf * NBH, NBH)], src_v)
                pltpu.sync_copy(seg_h.at[s, pl.ds(hf * NBH, NBH)], seg_v)

                def bstep(k, carry2):
                    gat = []
                    for j in range(NDEEP):
                        @pl.when(k > 0)
                        def _(j=j):
                            drain(j)
                        gat.append(pltpu.async_copy(
                            tab3.at[ci].at[src_v.at[k * NDEEP + j]],
                            rows_v.at[j], gsem.at[j]))
                    for j in range(NDEEP):
                        gat[j].wait()
                        pltpu.async_copy(
                            rows_v.at[j], acc.at[seg_v.at[k * NDEEP + j]],
                            ssem.at[j], add=True)
                    return carry2

                lax.fori_loop(0, NBH // NDEEP, bstep, 0)
                for j in range(NDEEP):
                    drain(j)
            plsc.subcore_barrier()
            pltpu.sync_copy(
                acc.at[pl.ds(s * STRIPE, STRIPE)],
                out_d.at[pl.ds(s * STRIPE, STRIPE), pl.ds(ci * 16, 16)])
        return carry

    lax.fori_loop(0, num_chunks, chunk_body, 0)

    if with_counts:
        @pl.when(c == (num_chunks % 2))
        def _():
            pltpu.sync_copy(zer_h, acc.at[pl.ds(s * STRIPE, STRIPE)])
            plsc.subcore_barrier()
            for hf in range(NHALF):
                pltpu.sync_copy(seg_h.at[s, pl.ds(hf * NBH, NBH)], seg_v)

                def cstep(k, carry2):
                    scat = [
                        pltpu.async_copy(
                            ones_v, acc.at[seg_v.at[k * NDEEP + j]],
                            ssem.at[j], add=True)
                        for j in range(NDEEP)]
                    for d in scat:
                        d.wait()
                    return carry2

                lax.fori_loop(0, NBH // NDEEP, cstep, 0)
            plsc.subcore_barrier()
            pltpu.sync_copy(acc.at[pl.ds(s * STRIPE, STRIPE)],
                            out_c.at[pl.ds(s * STRIPE, STRIPE)])

    return None


def _make_sc_agg(num_chunks, with_counts):
    mesh = plsc.VectorSubcoreMesh(core_axis_name="c", subcore_axis_name="s")
    out_type = (jax.ShapeDtypeStruct((SEG, num_chunks * 16), jnp.float32),)
    if with_counts:
        out_type = out_type + (jax.ShapeDtypeStruct((SEG, 16), jnp.float32),)
    scratch = [
        pltpu.VMEM((NBH, BATCH), jnp.int32),
        pltpu.VMEM((NBH, BATCH), jnp.int32),
        pltpu.VMEM((NDEEP, BATCH, 16), jnp.float32),
        pltpu.VMEM((BATCH, 16), jnp.float32),
        pltpu.VMEM_SHARED((SEG, 16), jnp.float32),
        pltpu.SemaphoreType.DMA((NDEEP,)),
        pltpu.SemaphoreType.DMA((NDEEP,)),
    ]
    return pl.kernel(
        functools.partial(_sc_agg_body, num_chunks, with_counts),
        out_type=out_type, mesh=mesh, scratch_types=scratch,
        compiler_params=pltpu.CompilerParams(use_tc_tiling_on_sc=False))


def _seg_body(et_ref, dst_ref, seg_ref):
    seg_ref[...] = et_ref[...] * N + dst_ref[...]


def _seg_ids(edge_type, dst):
    et = edge_type.reshape(2500, 128)
    d2 = dst.reshape(2500, 128)
    seg = pl.pallas_call(
        _seg_body,
        out_shape=jax.ShapeDtypeStruct((2500, 128), jnp.int32),
    )(et, d2)
    return seg.reshape(E)


def _h0_body(x_ref, ids_ref, emb_ref, h0_ref):
    ids = ids_ref[0, 0, :]
    oh = (ids[:, None] == lax.broadcasted_iota(jnp.int32, (BLK, NT), 1)
          ).astype(jnp.float32)
    h0_ref[:, :BASE] = x_ref[...]
    h0_ref[:, BASE:] = jnp.dot(oh, emb_ref[...],
                               preferred_element_type=jnp.float32)


def _h0_concat(x, node_type_ids, type_emb):
    ids3 = node_type_ids.reshape(NBK, 1, BLK)
    return pl.pallas_call(
        _h0_body,
        grid=(NBK,),
        in_specs=[
            pl.BlockSpec((BLK, BASE), lambda i: (i, 0)),
            pl.BlockSpec((1, 1, BLK), lambda i: (i, 0, 0)),
            pl.BlockSpec((NT, EMB), lambda i: (0, 0)),
        ],
        out_specs=pl.BlockSpec((BLK, BASE + EMB), lambda i: (i, 0)),
        out_shape=jax.ShapeDtypeStruct((N, BASE + EMB), jnp.float32),
    )(x, ids3, type_emb)


def _layer_body(D, h_ref, a_ref, cnt_ref, w_ref, root_ref, b_ref, out_ref):
    j = pl.program_id(1)

    @pl.when(j == 0)
    def _():
        out_ref[...] = jnp.zeros_like(out_ref)

    @pl.when(j < R)
    def _():
        inv = 1.0 / jnp.maximum(cnt_ref[:, 0], 1.0)
        a = a_ref[...] * inv[:, None]
        out_ref[...] += jnp.dot(a, w_ref[0],
                                preferred_element_type=jnp.float32)

    @pl.when(j == R)
    def _():
        acc = out_ref[...] + jnp.dot(h_ref[...], root_ref[...],
                                     preferred_element_type=jnp.float32)
        out_ref[...] = jnp.maximum(acc + b_ref[...], 0.0)


def _rgcn_dense(h, A, cnt, W, root, b):
    D = h.shape[1]
    return pl.pallas_call(
        functools.partial(_layer_body, D),
        grid=(NBK, R + 1),
        in_specs=[
            pl.BlockSpec((BLK, D), lambda i, j: (i, 0)),
            pl.BlockSpec((BLK, D),
                         lambda i, j: (jnp.minimum(j, R - 1) * NBK + i, 0)),
            pl.BlockSpec((BLK, 16),
                         lambda i, j: (jnp.minimum(j, R - 1) * NBK + i, 0)),
            pl.BlockSpec((1, D, HID), lambda i, j: (jnp.minimum(j, R - 1), 0, 0)),
            pl.BlockSpec((D, HID), lambda i, j: (0, 0)),
            pl.BlockSpec((1, HID), lambda i, j: (0, 0)),
        ],
        out_specs=pl.BlockSpec((BLK, HID), lambda i, j: (i, 0)),
        out_shape=jax.ShapeDtypeStruct((N, HID), jnp.float32),
    )(h, A, cnt, W, root, b.reshape(1, HID))


def _final_body(h_ref, batch_ref, nw_ref, nb_ref, gw_ref, gb_ref,
                node_ref, graph_ref, gsum_ref, gcnt_ref):
    i = pl.program_id(0)
    h = h_ref[...]
    node_ref[...] = jnp.dot(h, nw_ref[...],
                            preferred_element_type=jnp.float32) + nb_ref[0, 0]
    b = batch_ref[0, 0, :]
    oh = (b[:, None] == lax.broadcasted_iota(jnp.int32, (BLK, G), 1)
          ).astype(jnp.float32)
    gs = lax.dot_general(oh, h, (((0,), (0,)), ((), ())),
                         preferred_element_type=jnp.float32)
    gc = jnp.sum(oh, axis=0)[:, None]

    @pl.when(i == 0)
    def _():
        gsum_ref[...] = jnp.zeros_like(gsum_ref)
        gcnt_ref[...] = jnp.zeros_like(gcnt_ref)

    gsum_ref[...] += gs
    gcnt_ref[...] += jnp.broadcast_to(gc, gcnt_ref.shape)

    @pl.when(i == NBK - 1)
    def _():
        ge = gsum_ref[...] / jnp.maximum(gcnt_ref[...], 1.0)
        graph_ref[...] = jnp.dot(ge, gw_ref[...],
                                 preferred_element_type=jnp.float32) + gb_ref[0, 0]


def _heads(h2, batch, node_w, node_b, graph_w, graph_b):
    batch3 = batch.reshape(NBK, 1, BLK)
    node, graph = pl.pallas_call(
        _final_body,
        grid=(NBK,),
        in_specs=[
            pl.BlockSpec((BLK, HID), lambda i: (i, 0)),
            pl.BlockSpec((1, 1, BLK), lambda i: (i, 0, 0)),
            pl.BlockSpec((HID, 1), lambda i: (0, 0)),
            pl.BlockSpec((1, 1), lambda i: (0, 0)),
            pl.BlockSpec((HID, 1), lambda i: (0, 0)),
            pl.BlockSpec((1, 1), lambda i: (0, 0)),
        ],
        out_specs=[
            pl.BlockSpec((BLK, 1), lambda i: (i, 0)),
            pl.BlockSpec((G, 1), lambda i: (0, 0)),
        ],
        out_shape=[
            jax.ShapeDtypeStruct((N, 1), jnp.float32),
            jax.ShapeDtypeStruct((G, 1), jnp.float32),
        ],
        scratch_shapes=[
            pltpu.VMEM((G, HID), jnp.float32),
            pltpu.VMEM((G, HID), jnp.float32),
        ],
    )(h2, batch3, node_w, node_b.reshape(1, 1), graph_w,
      graph_b.reshape(1, 1))
    return node[:, 0], graph[:, 0]


_agg1 = _make_sc_agg((BASE + EMB) // 16, True)
_agg2 = _make_sc_agg(HID // 16, False)


def _chunk_tables(h):
    C = h.shape[1] // 16
    return h.reshape(N, C, 16).transpose(1, 0, 2)


def kernel(x, edge_index, edge_type, batch, node_type_ids, type_emb,
           W1, root1, b1, W2, root2, b2, node_w, node_b, graph_w, graph_b):
    src, dst = edge_index[0], edge_index[1]
    seg = _seg_ids(edge_type, dst)
    src3 = src.reshape(NTILES, NB, BATCH)
    seg3 = seg.reshape(NTILES, NB, BATCH)
    zeros = jnp.zeros((STRIPE, 16), jnp.float32)
    ones = jnp.ones((BATCH, 16), jnp.float32)

    h0 = _h0_concat(x, node_type_ids, type_emb)
    D0 = BASE + EMB

    A1, cnt = _agg1(_chunk_tables(h0), src3, seg3, zeros, ones)
    h1 = _rgcn_dense(h0, A1, cnt, W1, root1, b1)

    (A2,) = _agg2(_chunk_tables(h1), src3, seg3, zeros, ones)
    h2 = _rgcn_dense(h1, A2, cnt, W2, root2, b2)

    return _heads(h2, batch, node_w, node_b, graph_w, graph_b)
